# Initial kernel scaffold; baseline (speedup 1.0000x reference)
import jax
import jax.numpy as jnp
from jax.experimental import pallas as pl


def kernel(x, W):
    raise NotImplementedError("write your pallas kernel here")


# speedup vs baseline: 35.3110x; 35.3110x over previous
import jax
import jax.numpy as jnp
from jax import lax
from jax.experimental import pallas as pl
from jax.experimental.pallas import tpu as pltpu
from jax.experimental.pallas import tpu_sc as plsc
import functools

VOCAB = 1000
DIM = 16
CHARS = 20
NC = 2
NS = 16
NW = NC * NS


@functools.partial(jax.jit, static_argnames=("n_words", "chunk"))
def _sc_embed_max(x_flat, W, *, n_words, chunk):
    wpt = n_words // NW
    nch = wpt // chunk

    mesh = plsc.VectorSubcoreMesh(core_axis_name="c", subcore_axis_name="s")

    @functools.partial(
        pl.kernel,
        out_type=jax.ShapeDtypeStruct((n_words, DIM), jnp.float32),
        mesh=mesh,
        scratch_types=[
            pltpu.VMEM((VOCAB, DIM), jnp.float32),
            pltpu.VMEM((chunk, CHARS), jnp.int32),
            pltpu.VMEM((chunk, DIM), jnp.float32),
        ],
        compiler_params=pltpu.CompilerParams(
            needs_layout_passes=False, use_tc_tiling_on_sc=False),
    )
    def k(x_hbm, w_hbm, out_hbm, w_v, idx_v, out_v):
        wid = lax.axis_index("s") * NC + lax.axis_index("c")
        pltpu.sync_copy(w_hbm, w_v)
        col = lax.iota(jnp.int32, 16)

        for ch in range(nch):
            base = wid * wpt + ch * chunk
            pltpu.sync_copy(x_hbm.at[pl.ds(base, chunk)], idx_v)

            @pl.loop(0, chunk)
            def word_loop(w):
                iv0 = idx_v[w, pl.ds(0, 16)]
                iv1 = idx_v[w, pl.ds(4, 16)]
                scalars = [iv0[j] for j in range(16)]
                scalars += [iv1[j] for j in range(12, 16)]
                rows = [
                    plsc.load_gather(
                        w_v, [jnp.full((16,), s, jnp.int32), col])
                    for s in scalars
                ]
                while len(rows) > 1:
                    rows = [
                        jnp.maximum(rows[i], rows[i + 1])
                        if i + 1 < len(rows) else rows[i]
                        for i in range(0, len(rows), 2)
                    ]
                out_v[w, :] = rows[0]

            pltpu.sync_copy(out_v, out_hbm.at[pl.ds(base, chunk)])

    return k(x_flat, W)


def kernel(x, W):
    n_words = x.shape[0] * x.shape[1]
    x_flat = x.reshape(n_words, x.shape[2]).astype(jnp.int32)
    out = _sc_embed_max(x_flat, W, n_words=n_words, chunk=800)
    return out.reshape(x.shape[0], x.shape[1], DIM)


# speedup vs baseline: 35.3231x; 1.0003x over previous
import jax
import jax.numpy as jnp
from jax import lax
from jax.experimental import pallas as pl
from jax.experimental.pallas import tpu as pltpu
from jax.experimental.pallas import tpu_sc as plsc
import functools

VOCAB = 1000
DIM = 16
CHARS = 20
NC = 2
NS = 16
NW = NC * NS


@functools.partial(jax.jit, static_argnames=("n_words", "chunk"))
def _sc_embed_max(x_flat, W, *, n_words, chunk):
    wpt = n_words // NW
    nch = wpt // chunk

    mesh = plsc.VectorSubcoreMesh(core_axis_name="c", subcore_axis_name="s")

    @functools.partial(
        pl.kernel,
        out_type=jax.ShapeDtypeStruct((n_words, DIM), jnp.float32),
        mesh=mesh,
        scratch_types=[
            pltpu.VMEM((VOCAB, DIM), jnp.float32),
            pltpu.VMEM((chunk, CHARS), jnp.int32),
            pltpu.VMEM((chunk, DIM), jnp.float32),
        ],
        compiler_params=pltpu.CompilerParams(
            needs_layout_passes=False, use_tc_tiling_on_sc=False),
    )
    def k(x_hbm, w_hbm, out_hbm, w_v, idx_v, out_v):
        wid = lax.axis_index("s") * NC + lax.axis_index("c")
        pltpu.sync_copy(w_hbm, w_v)
        col = lax.iota(jnp.int32, 16)

        for ch in range(nch):
            base = wid * wpt + ch * chunk
            pltpu.sync_copy(x_hbm.at[pl.ds(base, chunk)], idx_v)

            @pl.loop(0, chunk)
            def word_loop(w):
                iv0 = idx_v[w, pl.ds(0, 16)]
                iv1 = idx_v[w, pl.ds(4, 16)]
                splats = [
                    jnp.take_along_axis(
                        iv0, jnp.full((16,), j, jnp.int32), axis=0,
                        mode="promise_in_bounds")
                    for j in range(16)
                ]
                splats += [
                    jnp.take_along_axis(
                        iv1, jnp.full((16,), j, jnp.int32), axis=0,
                        mode="promise_in_bounds")
                    for j in range(12, 16)
                ]
                rows = [plsc.load_gather(w_v, [s, col]) for s in splats]
                while len(rows) > 1:
                    rows = [
                        jnp.maximum(rows[i], rows[i + 1])
                        if i + 1 < len(rows) else rows[i]
                        for i in range(0, len(rows), 2)
                    ]
                out_v[w, :] = rows[0]

            pltpu.sync_copy(out_v, out_hbm.at[pl.ds(base, chunk)])

    return k(x_flat, W)


def kernel(x, W):
    n_words = x.shape[0] * x.shape[1]
    x_flat = x.reshape(n_words, x.shape[2]).astype(jnp.int32)
    out = _sc_embed_max(x_flat, W, n_words=n_words, chunk=800)
    return out.reshape(x.shape[0], x.shape[1], DIM)


# speedup vs baseline: 43.4806x; 1.2309x over previous
import jax
import jax.numpy as jnp
from jax import lax
from jax.experimental import pallas as pl
from jax.experimental.pallas import tpu as pltpu
from jax.experimental.pallas import tpu_sc as plsc
import functools

VOCAB = 1000
DIM = 16
CHARS = 20
NC = 2
NS = 16
NW = NC * NS


@functools.partial(jax.jit, static_argnames=("chunk",))
def _sc_embed_max(x, W, *, chunk):
    B, S, _ = x.shape
    rpt = B // NW
    nch = rpt // chunk

    mesh = plsc.VectorSubcoreMesh(core_axis_name="c", subcore_axis_name="s")

    @functools.partial(
        pl.kernel,
        out_type=jax.ShapeDtypeStruct((B, S, DIM), jnp.float32),
        mesh=mesh,
        scratch_types=[
            pltpu.VMEM((VOCAB, DIM), jnp.float32),
            pltpu.VMEM((chunk, S, CHARS), jnp.int32),
            pltpu.VMEM((chunk, S, DIM), jnp.float32),
        ],
        compiler_params=pltpu.CompilerParams(
            needs_layout_passes=False, use_tc_tiling_on_sc=False),
    )
    def k(x_hbm, w_hbm, out_hbm, w_v, idx_v, out_v):
        wid = lax.axis_index("s") * NC + lax.axis_index("c")
        pltpu.sync_copy(w_hbm, w_v)
        col = lax.iota(jnp.int32, 16)

        for ch in range(nch):
            base = wid * rpt + ch * chunk
            pltpu.sync_copy(x_hbm.at[pl.ds(base, chunk)], idx_v)

            for bb in range(chunk):

                @pl.loop(0, S)
                def word_loop(s):
                    iv0 = idx_v[bb, s, pl.ds(0, 16)]
                    iv1 = idx_v[bb, s, pl.ds(4, 16)]
                    splats = [
                        jnp.take_along_axis(
                            iv0, jnp.full((16,), j, jnp.int32), axis=0,
                            mode="promise_in_bounds")
                        for j in range(16)
                    ]
                    splats += [
                        jnp.take_along_axis(
                            iv1, jnp.full((16,), j, jnp.int32), axis=0,
                            mode="promise_in_bounds")
                        for j in range(12, 16)
                    ]
                    rows = [plsc.load_gather(w_v, [sp, col])
                            for sp in splats]
                    while len(rows) > 1:
                        rows = [
                            jnp.maximum(rows[i], rows[i + 1])
                            if i + 1 < len(rows) else rows[i]
                            for i in range(0, len(rows), 2)
                        ]
                    out_v[bb, s, :] = rows[0]

            pltpu.sync_copy(out_v, out_hbm.at[pl.ds(base, chunk)])

    return k(x, W)


def kernel(x, W):
    return _sc_embed_max(x.astype(jnp.int32), W, chunk=16)


# speedup vs baseline: 114.0951x; 2.6240x over previous
import jax
import jax.numpy as jnp
from jax import lax
from jax.experimental import pallas as pl
from jax.experimental.pallas import tpu as pltpu
from jax.experimental.pallas import tpu_sc as plsc
import functools

VOCAB = 1000
DIM = 16
CHARS = 20
NC = 2
NS = 16
NW = NC * NS
BT = 128


@functools.partial(jax.jit, static_argnames=("chunk",))
def _sc_embed_max(xt, W, *, chunk):
    _, S, B = xt.shape
    nch = S // chunk

    mesh = plsc.VectorSubcoreMesh(core_axis_name="c", subcore_axis_name="s")

    @functools.partial(
        pl.kernel,
        out_type=jax.ShapeDtypeStruct((S, DIM // 8, B // BT, 8, BT),
                                      jnp.float32),
        mesh=mesh,
        scratch_types=[
            pltpu.VMEM((VOCAB, DIM), jnp.float32),
            pltpu.VMEM((CHARS, chunk, BT), jnp.int32),
            pltpu.VMEM((chunk, DIM // 8, 1, 8, BT), jnp.float32),
            pltpu.VMEM((16, 17), jnp.float32),
        ],
        compiler_params=pltpu.CompilerParams(
            needs_layout_passes=False, use_tc_tiling_on_sc=False),
    )
    def k(x_hbm, w_hbm, out_hbm, w_v, idx_v, out_v, st_v):
        wid = lax.axis_index("s") * NC + lax.axis_index("c")
        pltpu.sync_copy(w_hbm, w_v)
        col = lax.iota(jnp.int32, 16)
        b0 = wid * BT

        @pl.loop(0, nch)
        def chunk_loop(ch):
            s0 = ch * chunk
            pltpu.sync_copy(
                x_hbm.at[:, pl.ds(s0, chunk), pl.ds(b0, BT)], idx_v)

            @pl.loop(0, chunk)
            def s_loop(s):

                @pl.loop(0, BT // 16)
                def blk_loop(l):
                    ivs = [idx_v[c, s, pl.ds(l * 16, 16)]
                           for c in range(CHARS)]
                    for j in range(16):
                        rows = []
                        for c in range(CHARS):
                            sp = jnp.take_along_axis(
                                ivs[c], jnp.full((16,), j, jnp.int32),
                                axis=0, mode="promise_in_bounds")
                            rows.append(plsc.load_gather(w_v, [sp, col]))
                        while len(rows) > 1:
                            rows = [
                                jnp.maximum(rows[i], rows[i + 1])
                                if i + 1 < len(rows) else rows[i]
                                for i in range(0, len(rows), 2)
                            ]
                        st_v[j, pl.ds(0, 16)] = rows[0]
                    for d in range(DIM):
                        tv = plsc.load_gather(
                            st_v, [col, jnp.full((16,), d, jnp.int32)])
                        out_v[s, d // 8, 0, d % 8, pl.ds(l * 16, 16)] = tv

            pltpu.sync_copy(
                out_v,
                out_hbm.at[pl.ds(s0, chunk), :, pl.ds(wid, 1)])

    return k(xt, W)


def kernel(x, W):
    B, S, _ = x.shape
    xt = jnp.transpose(x.astype(jnp.int32), (2, 1, 0))
    o = _sc_embed_max(xt, W, chunk=10)
    out = jnp.transpose(o, (2, 4, 0, 1, 3)).reshape(B, S, DIM)
    return out
